# Initial kernel scaffold; baseline (speedup 1.0000x reference)
import jax
import jax.numpy as jnp
from jax.experimental import pallas as pl


def kernel(features, corrupt_feat, edge_index, drop_mask, W0, b0, a0, W1, b1, a1, Wp, bp):
    raise NotImplementedError("write your pallas kernel here")


# speedup vs baseline: 2.3234x; 2.3234x over previous
import functools

import jax
import jax.numpy as jnp
from jax import lax
from jax.experimental import pallas as pl
from jax.experimental.pallas import tpu as pltpu
from jax.experimental.pallas import tpu_sc as plsc

N = 50000
E = 1600000
D = 256
NP = 50176
CB = 32
NBLK = 16
R = 1024
GRID = NP // R
NROW = NP // 128

NTILE = 16
NSC = 2
CHUNK = 80

_mesh = plsc.VectorSubcoreMesh(core_axis_name="c", subcore_axis_name="s")


def _deg_kernel(dst_hbm, degp_hbm, ones_v, didx, vbuf, shared):
    cid = lax.axis_index("c")
    sid = lax.axis_index("s")
    nloc = NP // NTILE

    def zinit(j, _):
        vbuf[pl.ds(j * 16, 16)] = jnp.zeros((16,), jnp.float32)
        return _

    lax.fori_loop(0, nloc // 16, zinit, None)

    def oinit(j, _):
        ones_v[pl.ds(j * 16, 16)] = jnp.ones((16,), jnp.float32)
        return _

    lax.fori_loop(0, CHUNK // 16, oinit, None)

    pltpu.sync_copy(vbuf, shared.at[pl.ds(sid * nloc, nloc)])
    plsc.subcore_barrier()

    ept = E // (NSC * NTILE)
    base = (cid * NTILE + sid) * ept

    def chunk(k, _):
        pltpu.sync_copy(dst_hbm.at[pl.ds(base + k * CHUNK, CHUNK)], didx)
        pltpu.sync_copy(ones_v, shared.at[didx], add=True)
        return _

    lax.fori_loop(0, ept // CHUNK, chunk, None)
    plsc.subcore_barrier()

    pltpu.sync_copy(shared.at[pl.ds(sid * nloc, nloc)], vbuf)
    pltpu.sync_copy(vbuf, degp_hbm.at[pl.ds(cid * NP + sid * nloc, nloc)])


def _deg_call(dst):
    k = functools.partial(
        pl.kernel,
        out_type=jax.ShapeDtypeStruct((NSC * NP,), jnp.float32),
        mesh=_mesh,
        compiler_params=pltpu.CompilerParams(use_tc_tiling_on_sc=False),
        scratch_types=[
            pltpu.VMEM((CHUNK,), jnp.float32),
            pltpu.VMEM((CHUNK,), jnp.int32),
            pltpu.VMEM((NP // NTILE,), jnp.float32),
            pltpu.VMEM_SHARED((NP,), jnp.float32),
        ],
    )(_deg_kernel)
    return k(dst)


NSB = 4
RNG = 4
RNGR = NP // RNG
RPT = RNGR // NTILE
ZR = 112


def _msg_kernel(src_hbm, dst_hbm, pd_hbm, m_hbm, zbuf, sidx, didx, rows, shared):
    cid = lax.axis_index("c")
    sid = lax.axis_index("s")

    def zinit(i, _):
        for jj in range(8):
            zbuf[i, pl.ds(jj * 16, 16)] = jnp.zeros((16,), jnp.float32)
        return _

    lax.fori_loop(0, ZR, zinit, None)

    ept = E // NTILE
    ebase = sid * ept
    lane = lax.iota(jnp.int32, 16)

    def per_pass(p, _):
        sb = cid * (NSB // NSC) + (p % (NSB // NSC))
        rng = p // (NSB // NSC)

        def zcp(i, _):
            pltpu.sync_copy(zbuf, shared.at[pl.ds(sid * RPT + i * ZR, ZR)])
            return _

        lax.fori_loop(0, RPT // ZR, zcp, None)
        plsc.subcore_barrier()

        def chunk(k, _):
            off = ebase + k * CHUNK
            pltpu.sync_copy(src_hbm.at[pl.ds(off, CHUNK)], sidx)
            pltpu.sync_copy(dst_hbm.at[pl.ds(off, CHUNK)], didx)

            def fixup(j, _):
                sl = pl.ds(j * 16, 16)
                sidx[sl] = sidx[sl] + sb * NP
                dloc = didx[sl] - rng * RNGR
                ok = (dloc >= 0) & (dloc < RNGR)
                didx[sl] = jnp.where(ok, dloc, RNGR + lane)
                return _

            lax.fori_loop(0, CHUNK // 16, fixup, None)
            pltpu.sync_copy(pd_hbm.at[sidx], rows)
            pltpu.sync_copy(rows, shared.at[didx], add=True)
            return _

        lax.fori_loop(0, ept // CHUNK, chunk, None)
        plsc.subcore_barrier()

        def wcp(i, _):
            roff = sid * RPT + i * ZR
            pltpu.sync_copy(shared.at[pl.ds(roff, ZR)], zbuf)
            pltpu.sync_copy(zbuf, m_hbm.at[pl.ds(sb * NP + rng * RNGR + roff, ZR)])
            return _

        lax.fori_loop(0, RPT // ZR, wcp, None)
        plsc.subcore_barrier()
        return _

    lax.fori_loop(0, (NSB // NSC) * RNG, per_pass, None)


def _msg_call(src, dst, pd_flat):
    k = functools.partial(
        pl.kernel,
        out_type=jax.ShapeDtypeStruct((NSB * NP, 128), jnp.float32),
        mesh=_mesh,
        compiler_params=pltpu.CompilerParams(use_tc_tiling_on_sc=False),
        scratch_types=[
            pltpu.VMEM((ZR, 128), jnp.float32),
            pltpu.VMEM((CHUNK,), jnp.int32),
            pltpu.VMEM((CHUNK,), jnp.int32),
            pltpu.VMEM((CHUNK, 128), jnp.float32),
            pltpu.VMEM_SHARED((RNGR + 16, 128), jnp.float32),
        ],
    )(_msg_kernel)
    return k(src, dst, pd_flat)


def _scal_kernel(src_hbm, dst_hbm, u1_hbm, u2_hbm, tp_hbm,
                 sidx, didx, r1, r2, vbuf, sh1, sh2):
    cid = lax.axis_index("c")
    sid = lax.axis_index("s")
    nloc = NP // NTILE

    def zinit(j, _):
        vbuf[pl.ds(j * 16, 16)] = jnp.zeros((16,), jnp.float32)
        return _

    lax.fori_loop(0, nloc // 16, zinit, None)
    pltpu.sync_copy(vbuf, sh1.at[pl.ds(sid * nloc, nloc)])
    pltpu.sync_copy(vbuf, sh2.at[pl.ds(sid * nloc, nloc)])
    plsc.subcore_barrier()

    ept = E // (NSC * NTILE)
    base = (cid * NTILE + sid) * ept

    def chunk(k, _):
        off = base + k * CHUNK
        pltpu.sync_copy(src_hbm.at[pl.ds(off, CHUNK)], sidx)
        pltpu.sync_copy(dst_hbm.at[pl.ds(off, CHUNK)], didx)
        pltpu.sync_copy(u1_hbm.at[sidx], r1)
        pltpu.sync_copy(u2_hbm.at[sidx], r2)
        pltpu.sync_copy(r1, sh1.at[didx], add=True)
        pltpu.sync_copy(r2, sh2.at[didx], add=True)
        return _

    lax.fori_loop(0, ept // CHUNK, chunk, None)
    plsc.subcore_barrier()

    pltpu.sync_copy(sh1.at[pl.ds(sid * nloc, nloc)], vbuf)
    pltpu.sync_copy(vbuf, tp_hbm.at[pl.ds((cid * 2 + 0) * NP + sid * nloc, nloc)])
    pltpu.sync_copy(sh2.at[pl.ds(sid * nloc, nloc)], vbuf)
    pltpu.sync_copy(vbuf, tp_hbm.at[pl.ds((cid * 2 + 1) * NP + sid * nloc, nloc)])


def _scal_call(src, dst, u1, u2):
    k = functools.partial(
        pl.kernel,
        out_type=jax.ShapeDtypeStruct((NSC * 2 * NP,), jnp.float32),
        mesh=_mesh,
        compiler_params=pltpu.CompilerParams(use_tc_tiling_on_sc=False),
        scratch_types=[
            pltpu.VMEM((CHUNK,), jnp.int32),
            pltpu.VMEM((CHUNK,), jnp.int32),
            pltpu.VMEM((CHUNK,), jnp.float32),
            pltpu.VMEM((CHUNK,), jnp.float32),
            pltpu.VMEM((NP // NTILE,), jnp.float32),
            pltpu.VMEM_SHARED((NP,), jnp.float32),
            pltpu.VMEM_SHARED((NP,), jnp.float32),
        ],
    )(_scal_kernel)
    return k(src, dst, u1, u2)


def _tc1_kernel(x1_ref, x2_ref, dm_ref, w0_ref, dg0_ref, dg1_ref, pd_ref, dis_ref):
    w0p = dm_ref[...][:, None] * w0_ref[...]
    p1 = jnp.dot(x1_ref[...], w0p, preferred_element_type=jnp.float32)
    p2 = jnp.dot(x2_ref[...], w0p, preferred_element_type=jnp.float32)
    deg = dg0_ref[...] + dg1_ref[...]
    dis2 = jnp.where(deg > 0.0, lax.rsqrt(jnp.maximum(deg, 1e-12)), 0.0)
    dis_ref[...] = dis2
    for sb in range(NSB):
        cols = []
        for q in range(NSB):
            c = sb * NSB + q
            p = p1 if c < NBLK // 2 else p2
            cc = c % (NBLK // 2)
            cols.append(p[:, cc * CB:(cc + 1) * CB])
        pd_ref[sb, :, :] = jnp.concatenate(cols, axis=1) * dis2


def _tc1_call(x1, x2, dm, w0, dg0, dg1):
    return pl.pallas_call(
        _tc1_kernel,
        grid=(GRID,),
        in_specs=[
            pl.BlockSpec((R, D), lambda i: (i, 0)),
            pl.BlockSpec((R, D), lambda i: (i, 0)),
            pl.BlockSpec((D,), lambda i: (0,)),
            pl.BlockSpec((D, D), lambda i: (0, 0)),
            pl.BlockSpec((R, 1), lambda i: (i, 0)),
            pl.BlockSpec((R, 1), lambda i: (i, 0)),
        ],
        out_specs=[
            pl.BlockSpec((NSB, R, 128), lambda i: (0, i, 0)),
            pl.BlockSpec((R, 1), lambda i: (i, 0)),
        ],
        out_shape=[
            jax.ShapeDtypeStruct((NSB, NP, 128), jnp.float32),
            jax.ShapeDtypeStruct((NP, 1), jnp.float32),
        ],
    )(x1, x2, dm, w0, dg0, dg1)


def _tc2_kernel(m_ref, dis_ref, b0_ref, a0_ref, w1_ref, wp_ref, u1_ref, u2_ref):
    wp_vec = jnp.sum(wp_ref[...], axis=1, keepdims=True)
    w1p = jnp.dot(w1_ref[...], wp_vec, preferred_element_type=jnp.float32)
    dis2 = dis_ref[...]
    a0 = a0_ref[0]
    b0 = b0_ref[...]
    m = m_ref[...]
    halves = []
    for h in range(2):
        gs = []
        for cc in range(NBLK // 2):
            c = h * (NBLK // 2) + cc
            sb, q = c // NSB, c % NSB
            z = m[sb, :, q * CB:(q + 1) * CB] * dis2 + b0[cc * CB:(cc + 1) * CB]
            gs.append(jnp.where(z >= 0.0, z, a0 * z))
        g = jnp.concatenate(gs, axis=1)
        halves.append(jnp.dot(g, w1p, preferred_element_type=jnp.float32))
    u1_ref[...] = halves[0] * dis2
    u2_ref[...] = halves[1] * dis2


def _tc2_call(m_blocked, dis, b0, a0, w1, wp):
    return pl.pallas_call(
        _tc2_kernel,
        grid=(GRID,),
        in_specs=[
            pl.BlockSpec((NSB, R, 128), lambda i: (0, i, 0)),
            pl.BlockSpec((R, 1), lambda i: (i, 0)),
            pl.BlockSpec((D,), lambda i: (0,)),
            pl.BlockSpec((1,), lambda i: (0,)),
            pl.BlockSpec((D, D), lambda i: (0, 0)),
            pl.BlockSpec((D, D), lambda i: (0, 0)),
        ],
        out_specs=[
            pl.BlockSpec((R, 1), lambda i: (i, 0)),
            pl.BlockSpec((R, 1), lambda i: (i, 0)),
        ],
        out_shape=[
            jax.ShapeDtypeStruct((NP, 1), jnp.float32),
            jax.ShapeDtypeStruct((NP, 1), jnp.float32),
        ],
    )(m_blocked, dis, b0, a0, w1, wp)


def _tc3_kernel(t00_ref, t01_ref, t10_ref, t11_ref, dis_ref, b1_ref, wp_ref,
                bp_ref, out_ref):
    i = pl.program_id(0)
    wp_vec = jnp.sum(wp_ref[...], axis=1)
    cst = jnp.sum(b1_ref[...] * wp_vec) + jnp.sum(bp_ref[...])
    dis = dis_ref[...]
    s1 = dis * (t00_ref[...] + t10_ref[...]) + cst
    s2 = dis * (t01_ref[...] + t11_ref[...]) + cst
    l1 = jnp.maximum(-s1, 0.0) + jnp.log1p(jnp.exp(-jnp.abs(s1)))
    l2 = jnp.maximum(s2, 0.0) + jnp.log1p(jnp.exp(-jnp.abs(s2)))
    nid = i * R + lax.broadcasted_iota(jnp.int32, (R, 1), 0)
    valid = nid < N
    part = jnp.sum(jnp.where(valid, l1 + l2, 0.0))

    @pl.when(i == 0)
    def _():
        out_ref[0, 0] = 0.0

    out_ref[0, 0] += part

    @pl.when(i == GRID - 1)
    def _():
        out_ref[0, 0] = out_ref[0, 0] * (1.0 / (2.0 * N))


def _tc3_call(tp, dis, b1, wp, bp):
    return pl.pallas_call(
        _tc3_kernel,
        grid=(GRID,),
        in_specs=[
            pl.BlockSpec((R, 1), lambda i: (i, 0)),
            pl.BlockSpec((R, 1), lambda i: (i, 0)),
            pl.BlockSpec((R, 1), lambda i: (i, 0)),
            pl.BlockSpec((R, 1), lambda i: (i, 0)),
            pl.BlockSpec((R, 1), lambda i: (i, 0)),
            pl.BlockSpec((D,), lambda i: (0,)),
            pl.BlockSpec((D, D), lambda i: (0, 0)),
            pl.BlockSpec((D,), lambda i: (0,)),
        ],
        out_specs=pl.BlockSpec((1, 1), lambda i: (0, 0),
                               memory_space=pltpu.SMEM),
        out_shape=jax.ShapeDtypeStruct((1, 1), jnp.float32),
    )(tp[0, 0], tp[0, 1], tp[1, 0], tp[1, 1], dis, b1, wp, bp)


def kernel(features, corrupt_feat, edge_index, drop_mask, W0, b0, a0, W1, b1, a1, Wp, bp):
    src = edge_index[0].astype(jnp.int32)
    dst = edge_index[1].astype(jnp.int32)

    USE_SC_A, USE_SC_B, USE_SC_C = True, True, True
    if USE_SC_A:
        degp = _deg_call(dst).reshape(NSC, NP, 1)
    else:
        d0 = jnp.zeros((NP,), jnp.float32).at[dst].add(1.0)
        degp = jnp.stack([d0, jnp.zeros((NP,), jnp.float32)]).reshape(NSC, NP, 1)
    pd, dis = _tc1_call(features, corrupt_feat, drop_mask, W0, degp[0], degp[1])
    if USE_SC_B:
        m_flat = _msg_call(src, dst, pd.reshape(NSB * NP, 128))
    else:
        pdf = pd.reshape(NSB * NP, 128)
        m_flat = jnp.zeros((NSB * NP, 128), jnp.float32)
        for b in range(NSB):
            acc = jnp.zeros((NP, 128), jnp.float32).at[dst].add(pdf[b * NP:(b + 1) * NP][src])
            m_flat = m_flat.at[b * NP:(b + 1) * NP].set(acc)
    u1, u2 = _tc2_call(m_flat.reshape(NSB, NP, 128), dis, b0, a0, W1, Wp)
    if USE_SC_C:
        tp = _scal_call(src, dst, u1.reshape(NP), u2.reshape(NP))
        tp = tp.reshape(NSC, 2, NP, 1)
    else:
        t1 = jnp.zeros((NP,), jnp.float32).at[dst].add(u1.reshape(NP)[src])
        t2 = jnp.zeros((NP,), jnp.float32).at[dst].add(u2.reshape(NP)[src])
        z = jnp.zeros((NP,), jnp.float32)
        tp = jnp.stack([t1, t2, z, z]).reshape(NSC, 2, NP, 1)
    loss = _tc3_call(tp, dis, b1, Wp, bp)
    return loss[0, 0]


# speedup vs baseline: 2.8805x; 1.2398x over previous
import functools

import jax
import jax.numpy as jnp
from jax import lax
from jax.experimental import pallas as pl
from jax.experimental.pallas import tpu as pltpu
from jax.experimental.pallas import tpu_sc as plsc

N = 50000
E = 1600000
D = 256
NP = 50176
CB = 32
NBLK = 16
R = 1024
GRID = NP // R
NROW = NP // 128

NTILE = 16
NSC = 2
CHUNK = 80

_mesh = plsc.VectorSubcoreMesh(core_axis_name="c", subcore_axis_name="s")


def _deg_kernel(dst_hbm, degp_hbm, ones_v, didx, vbuf, shared):
    cid = lax.axis_index("c")
    sid = lax.axis_index("s")
    nloc = NP // NTILE

    def zinit(j, _):
        vbuf[pl.ds(j * 16, 16)] = jnp.zeros((16,), jnp.float32)
        return _

    lax.fori_loop(0, nloc // 16, zinit, None)

    def oinit(j, _):
        ones_v[pl.ds(j * 16, 16)] = jnp.ones((16,), jnp.float32)
        return _

    lax.fori_loop(0, CHUNK // 16, oinit, None)

    pltpu.sync_copy(vbuf, shared.at[pl.ds(sid * nloc, nloc)])
    plsc.subcore_barrier()

    ept = E // (NSC * NTILE)
    base = (cid * NTILE + sid) * ept

    def chunk(k, _):
        pltpu.sync_copy(dst_hbm.at[pl.ds(base + k * CHUNK, CHUNK)], didx)
        pltpu.sync_copy(ones_v, shared.at[didx], add=True)
        return _

    lax.fori_loop(0, ept // CHUNK, chunk, None)
    plsc.subcore_barrier()

    pltpu.sync_copy(shared.at[pl.ds(sid * nloc, nloc)], vbuf)
    pltpu.sync_copy(vbuf, degp_hbm.at[pl.ds(cid * NP + sid * nloc, nloc)])


def _deg_call(dst):
    k = functools.partial(
        pl.kernel,
        out_type=jax.ShapeDtypeStruct((NSC * NP,), jnp.float32),
        mesh=_mesh,
        compiler_params=pltpu.CompilerParams(use_tc_tiling_on_sc=False),
        scratch_types=[
            pltpu.VMEM((CHUNK,), jnp.float32),
            pltpu.VMEM((CHUNK,), jnp.int32),
            pltpu.VMEM((NP // NTILE,), jnp.float32),
            pltpu.VMEM_SHARED((NP,), jnp.float32),
        ],
    )(_deg_kernel)
    return k(dst)


NSB = 4
RNG = 4
RNGR = NP // RNG
RPT = RNGR // NTILE
ZR = 56


def _msg_kernel(src_hbm, dst_hbm, pd_hbm, m_hbm, zbuf, sidx, didx, rows,
                shared):
    cid = lax.axis_index("c")
    sid = lax.axis_index("s")

    def zinit(i, _):
        for jj in range(8):
            zbuf[i, pl.ds(jj * 16, 16)] = jnp.zeros((16,), jnp.float32)
        return _

    lax.fori_loop(0, ZR, zinit, None)

    ept = E // NTILE
    ebase = sid * ept
    lane = lax.iota(jnp.int32, 16)

    def per_pass(p, _):
        sb = cid * (NSB // NSC) + (p % (NSB // NSC))
        rng = p // (NSB // NSC)

        def zcp(i, _):
            pltpu.sync_copy(zbuf, shared.at[pl.ds(sid * RPT + i * ZR, ZR)])
            return _

        lax.fori_loop(0, RPT // ZR, zcp, None)
        plsc.subcore_barrier()

        def chunk(k, _):
            off = ebase + k * CHUNK
            pltpu.sync_copy(src_hbm.at[pl.ds(off, CHUNK)], sidx)
            pltpu.sync_copy(dst_hbm.at[pl.ds(off, CHUNK)], didx)

            def fixup(j, _):
                sl = pl.ds(j * 16, 16)
                sidx[sl] = sidx[sl] + sb * NP
                dloc = didx[sl] - rng * RNGR
                ok = (dloc >= 0) & (dloc < RNGR)
                didx[sl] = jnp.where(ok, dloc, RNGR + lane)
                return _

            lax.fori_loop(0, CHUNK // 16, fixup, None)
            pltpu.sync_copy(pd_hbm.at[sidx], rows)
            pltpu.sync_copy(rows, shared.at[didx], add=True)
            return _

        lax.fori_loop(0, ept // CHUNK, chunk, None)
        plsc.subcore_barrier()

        def wcp(i, _):
            roff = sid * RPT + i * ZR
            pltpu.sync_copy(shared.at[pl.ds(roff, ZR)], zbuf)
            pltpu.sync_copy(zbuf, m_hbm.at[pl.ds(sb * NP + rng * RNGR + roff, ZR)])
            return _

        lax.fori_loop(0, RPT // ZR, wcp, None)
        plsc.subcore_barrier()
        return _

    lax.fori_loop(0, (NSB // NSC) * RNG, per_pass, None)


def _msg_call(src, dst, pd_flat):
    k = functools.partial(
        pl.kernel,
        out_type=jax.ShapeDtypeStruct((NSB * NP, 128), jnp.float32),
        mesh=_mesh,
        compiler_params=pltpu.CompilerParams(use_tc_tiling_on_sc=False),
        scratch_types=[
            pltpu.VMEM((ZR, 128), jnp.float32),
            pltpu.VMEM((CHUNK,), jnp.int32),
            pltpu.VMEM((CHUNK,), jnp.int32),
            pltpu.VMEM((CHUNK, 128), jnp.float32),
            pltpu.VMEM_SHARED((RNGR + 16, 128), jnp.float32),
        ],
    )(_msg_kernel)
    return k(src, dst, pd_flat)


def _scal_kernel(src_hbm, dst_hbm, u1_hbm, u2_hbm, tp_hbm,
                 sidx, didx, r1, r2, vbuf, sh1, sh2):
    cid = lax.axis_index("c")
    sid = lax.axis_index("s")
    nloc = NP // NTILE

    def zinit(j, _):
        vbuf[pl.ds(j * 16, 16)] = jnp.zeros((16,), jnp.float32)
        return _

    lax.fori_loop(0, nloc // 16, zinit, None)
    pltpu.sync_copy(vbuf, sh1.at[pl.ds(sid * nloc, nloc)])
    pltpu.sync_copy(vbuf, sh2.at[pl.ds(sid * nloc, nloc)])
    plsc.subcore_barrier()

    ept = E // (NSC * NTILE)
    base = (cid * NTILE + sid) * ept

    def chunk(k, _):
        off = base + k * CHUNK
        pltpu.sync_copy(src_hbm.at[pl.ds(off, CHUNK)], sidx)
        pltpu.sync_copy(dst_hbm.at[pl.ds(off, CHUNK)], didx)
        pltpu.sync_copy(u1_hbm.at[sidx], r1)
        pltpu.sync_copy(u2_hbm.at[sidx], r2)
        pltpu.sync_copy(r1, sh1.at[didx], add=True)
        pltpu.sync_copy(r2, sh2.at[didx], add=True)
        return _

    lax.fori_loop(0, ept // CHUNK, chunk, None)
    plsc.subcore_barrier()

    pltpu.sync_copy(sh1.at[pl.ds(sid * nloc, nloc)], vbuf)
    pltpu.sync_copy(vbuf, tp_hbm.at[pl.ds((cid * 2 + 0) * NP + sid * nloc, nloc)])
    pltpu.sync_copy(sh2.at[pl.ds(sid * nloc, nloc)], vbuf)
    pltpu.sync_copy(vbuf, tp_hbm.at[pl.ds((cid * 2 + 1) * NP + sid * nloc, nloc)])


def _scal_call(src, dst, u1, u2):
    k = functools.partial(
        pl.kernel,
        out_type=jax.ShapeDtypeStruct((NSC * 2 * NP,), jnp.float32),
        mesh=_mesh,
        compiler_params=pltpu.CompilerParams(use_tc_tiling_on_sc=False),
        scratch_types=[
            pltpu.VMEM((CHUNK,), jnp.int32),
            pltpu.VMEM((CHUNK,), jnp.int32),
            pltpu.VMEM((CHUNK,), jnp.float32),
            pltpu.VMEM((CHUNK,), jnp.float32),
            pltpu.VMEM((NP // NTILE,), jnp.float32),
            pltpu.VMEM_SHARED((NP,), jnp.float32),
            pltpu.VMEM_SHARED((NP,), jnp.float32),
        ],
    )(_scal_kernel)
    return k(src, dst, u1, u2)


def _tc1_kernel(x1_ref, x2_ref, dm_ref, w0_ref, dg0_ref, dg1_ref, pd_ref, dis_ref):
    w0p = dm_ref[...][:, None] * w0_ref[...]
    p1 = jnp.dot(x1_ref[...], w0p, preferred_element_type=jnp.float32)
    p2 = jnp.dot(x2_ref[...], w0p, preferred_element_type=jnp.float32)
    deg = dg0_ref[...] + dg1_ref[...]
    dis2 = jnp.where(deg > 0.0, lax.rsqrt(jnp.maximum(deg, 1e-12)), 0.0)
    dis_ref[...] = dis2
    for sb in range(NSB):
        cols = []
        for q in range(NSB):
            c = sb * NSB + q
            p = p1 if c < NBLK // 2 else p2
            cc = c % (NBLK // 2)
            cols.append(p[:, cc * CB:(cc + 1) * CB])
        pd_ref[sb, :, :] = jnp.concatenate(cols, axis=1) * dis2


def _tc1_call(x1, x2, dm, w0, dg0, dg1):
    return pl.pallas_call(
        _tc1_kernel,
        grid=(GRID,),
        in_specs=[
            pl.BlockSpec((R, D), lambda i: (i, 0)),
            pl.BlockSpec((R, D), lambda i: (i, 0)),
            pl.BlockSpec((D,), lambda i: (0,)),
            pl.BlockSpec((D, D), lambda i: (0, 0)),
            pl.BlockSpec((R, 1), lambda i: (i, 0)),
            pl.BlockSpec((R, 1), lambda i: (i, 0)),
        ],
        out_specs=[
            pl.BlockSpec((NSB, R, 128), lambda i: (0, i, 0)),
            pl.BlockSpec((R, 1), lambda i: (i, 0)),
        ],
        out_shape=[
            jax.ShapeDtypeStruct((NSB, NP, 128), jnp.float32),
            jax.ShapeDtypeStruct((NP, 1), jnp.float32),
        ],
    )(x1, x2, dm, w0, dg0, dg1)


def _tc2_kernel(m_ref, dis_ref, b0_ref, a0_ref, w1_ref, wp_ref, u1_ref, u2_ref):
    wp_vec = jnp.sum(wp_ref[...], axis=1, keepdims=True)
    w1p = jnp.dot(w1_ref[...], wp_vec, preferred_element_type=jnp.float32)
    dis2 = dis_ref[...]
    a0 = a0_ref[0]
    b0 = b0_ref[...]
    m = m_ref[...]
    halves = []
    for h in range(2):
        gs = []
        for cc in range(NBLK // 2):
            c = h * (NBLK // 2) + cc
            sb, q = c // NSB, c % NSB
            z = m[sb, :, q * CB:(q + 1) * CB] * dis2 + b0[cc * CB:(cc + 1) * CB]
            gs.append(jnp.where(z >= 0.0, z, a0 * z))
        g = jnp.concatenate(gs, axis=1)
        halves.append(jnp.dot(g, w1p, preferred_element_type=jnp.float32))
    u1_ref[...] = halves[0] * dis2
    u2_ref[...] = halves[1] * dis2


def _tc2_call(m_blocked, dis, b0, a0, w1, wp):
    return pl.pallas_call(
        _tc2_kernel,
        grid=(GRID,),
        in_specs=[
            pl.BlockSpec((NSB, R, 128), lambda i: (0, i, 0)),
            pl.BlockSpec((R, 1), lambda i: (i, 0)),
            pl.BlockSpec((D,), lambda i: (0,)),
            pl.BlockSpec((1,), lambda i: (0,)),
            pl.BlockSpec((D, D), lambda i: (0, 0)),
            pl.BlockSpec((D, D), lambda i: (0, 0)),
        ],
        out_specs=[
            pl.BlockSpec((R, 1), lambda i: (i, 0)),
            pl.BlockSpec((R, 1), lambda i: (i, 0)),
        ],
        out_shape=[
            jax.ShapeDtypeStruct((NP, 1), jnp.float32),
            jax.ShapeDtypeStruct((NP, 1), jnp.float32),
        ],
    )(m_blocked, dis, b0, a0, w1, wp)


def _tc3_kernel(t00_ref, t01_ref, t10_ref, t11_ref, dis_ref, b1_ref, wp_ref,
                bp_ref, out_ref):
    i = pl.program_id(0)
    wp_vec = jnp.sum(wp_ref[...], axis=1)
    cst = jnp.sum(b1_ref[...] * wp_vec) + jnp.sum(bp_ref[...])
    dis = dis_ref[...]
    s1 = dis * (t00_ref[...] + t10_ref[...]) + cst
    s2 = dis * (t01_ref[...] + t11_ref[...]) + cst
    l1 = jnp.maximum(-s1, 0.0) + jnp.log1p(jnp.exp(-jnp.abs(s1)))
    l2 = jnp.maximum(s2, 0.0) + jnp.log1p(jnp.exp(-jnp.abs(s2)))
    nid = i * R + lax.broadcasted_iota(jnp.int32, (R, 1), 0)
    valid = nid < N
    part = jnp.sum(jnp.where(valid, l1 + l2, 0.0))

    @pl.when(i == 0)
    def _():
        out_ref[0, 0] = 0.0

    out_ref[0, 0] += part

    @pl.when(i == GRID - 1)
    def _():
        out_ref[0, 0] = out_ref[0, 0] * (1.0 / (2.0 * N))


def _tc3_call(tp, dis, b1, wp, bp):
    return pl.pallas_call(
        _tc3_kernel,
        grid=(GRID,),
        in_specs=[
            pl.BlockSpec((R, 1), lambda i: (i, 0)),
            pl.BlockSpec((R, 1), lambda i: (i, 0)),
            pl.BlockSpec((R, 1), lambda i: (i, 0)),
            pl.BlockSpec((R, 1), lambda i: (i, 0)),
            pl.BlockSpec((R, 1), lambda i: (i, 0)),
            pl.BlockSpec((D,), lambda i: (0,)),
            pl.BlockSpec((D, D), lambda i: (0, 0)),
            pl.BlockSpec((D,), lambda i: (0,)),
        ],
        out_specs=pl.BlockSpec((1, 1), lambda i: (0, 0),
                               memory_space=pltpu.SMEM),
        out_shape=jax.ShapeDtypeStruct((1, 1), jnp.float32),
    )(tp[0, 0], tp[0, 1], tp[1, 0], tp[1, 1], dis, b1, wp, bp)


def kernel(features, corrupt_feat, edge_index, drop_mask, W0, b0, a0, W1, b1, a1, Wp, bp):
    src = edge_index[0].astype(jnp.int32)
    dst = edge_index[1].astype(jnp.int32)

    degp = _deg_call(dst).reshape(NSC, NP, 1)
    pd, dis = _tc1_call(features, corrupt_feat, drop_mask, W0, degp[0], degp[1])
    pdf = pd.reshape(NSB * NP, 128)
    macc = [jnp.zeros((NP, 128), jnp.float32).at[dst].add(
        pdf[b * NP:(b + 1) * NP][src]) for b in range(NSB)]
    m_flat = jnp.concatenate(macc, axis=0)
    u1, u2 = _tc2_call(m_flat.reshape(NSB, NP, 128), dis, b0, a0, W1, Wp)
    tp = _scal_call(src, dst, u1.reshape(NP), u2.reshape(NP))
    tp = tp.reshape(NSC, 2, NP, 1)
    loss = _tc3_call(tp, dis, b1, Wp, bp)
    return loss[0, 0]
